# baseline (speedup 1.0000x reference)
import functools

import jax
import jax.numpy as jnp
from jax.experimental import pallas as pl
from jax.experimental.pallas import tpu as pltpu


def _fused_kernel(x_ref, c1_ref, c1b_ref, sl_ref, sr_ref, c2_ref, c2b_ref,
                  w1_ref, b1_ref, w2_ref, b2_ref, o_ref):
    B = x_ref.shape[1]
    xf = x_ref[...].reshape(28 * B, 28)

    y1 = c1b_ref[...] + sum(
        jnp.dot(xf[ki * B:(ki + 24) * B], c1_ref[ki],
                preferred_element_type=jnp.float32)
        for ki in range(5))
    y1 = jnp.maximum(y1, 0.0).astype(jnp.bfloat16)

    mh = jnp.maximum(y1[0:23 * B], y1[B:24 * B])
    mhw = jnp.maximum(mh[:, 0:239], mh[:, 1:240])

    n = jnp.dot(mhw, sr_ref[...],
                preferred_element_type=jnp.float32)
    n4 = n.astype(jnp.bfloat16).reshape(23, B, 120)

    p = jnp.concatenate(
        [sum(n4[h] * sl_ref[i, h].astype(jnp.bfloat16) for h in range(23))
         for i in range(12)], axis=0)

    acc2 = c2b_ref[...] + sum(
        jnp.dot(p[ki * B:(ki + 10) * B], c2_ref[ki],
                preferred_element_type=jnp.float32)
        for ki in range(3))
    y2 = jnp.maximum(acc2, 0.0).astype(jnp.bfloat16)

    acc = b1_ref[...] + sum(
        jnp.dot(y2[h * B:(h + 1) * B], w1_ref[h],
                preferred_element_type=jnp.float32)
        for h in range(10))
    h1 = jnp.maximum(acc, 0.0).astype(jnp.bfloat16)

    z = jnp.dot(h1, w2_ref[...], preferred_element_type=jnp.float32) \
        + b2_ref[...]
    m = jnp.max(z, axis=-1, keepdims=True)
    lse = jnp.log(jnp.sum(jnp.exp(z - m), axis=-1, keepdims=True)) + m
    o_ref[...] = z - lse


@functools.partial(jax.jit, static_argnames=())
def kernel(x, conv1_band, conv1_bias, pool_sl, pool_sr, conv2_band,
           conv2_bias, fc1_w, fc1_b, fc2_w, fc2_b):
    N = x.shape[0]

    B = 64
    while N % B:
        B //= 2
    grid = N // B

    xt = x.reshape(N, 28, 28).astype(jnp.bfloat16).transpose(1, 0, 2)
    c1 = conv1_band.astype(jnp.bfloat16)
    sr = pool_sr.astype(jnp.bfloat16)
    c2 = conv2_band.astype(jnp.bfloat16)
    w1 = fc1_w.reshape(10, 200, 500).astype(jnp.bfloat16)
    w2 = fc2_w.astype(jnp.bfloat16)

    return pl.pallas_call(
        _fused_kernel,
        out_shape=jax.ShapeDtypeStruct((N, 10), jnp.float32),
        grid=(grid,),
        in_specs=[
            pl.BlockSpec((28, B, 28), lambda b: (0, b, 0)),
            pl.BlockSpec((5, 28, 240), lambda b: (0, 0, 0)),
            pl.BlockSpec((1, 240), lambda b: (0, 0)),
            pl.BlockSpec(memory_space=pltpu.SMEM),
            pl.BlockSpec((239, 120), lambda b: (0, 0)),
            pl.BlockSpec((3, 120, 200), lambda b: (0, 0, 0)),
            pl.BlockSpec((1, 200), lambda b: (0, 0)),
            pl.BlockSpec((10, 200, 500), lambda b: (0, 0, 0)),
            pl.BlockSpec((1, 500), lambda b: (0, 0)),
            pl.BlockSpec((500, 10), lambda b: (0, 0)),
            pl.BlockSpec((1, 10), lambda b: (0, 0)),
        ],
        out_specs=pl.BlockSpec((B, 10), lambda b: (b, 0)),
        compiler_params=pltpu.CompilerParams(
            dimension_semantics=("parallel",)),
        cost_estimate=pl.CostEstimate(
            flops=N * (24 * 140 * 240 + 23 * 239 * 120 + 10 * 360 * 200
                       + 2000 * 500 + 500 * 10) * 2,
            transcendentals=N * 11,
            bytes_accessed=N * (784 * 2 + 40) + 2 * (140 * 240 + 239 * 120
                                                     + 360 * 200 + 2000 * 500
                                                     + 500 * 10),
        ),
    )(xt, c1, conv1_bias, pool_sl, sr, c2, conv2_bias, w1, fc1_b,
      w2, fc2_b)


# speedup vs baseline: 1.8644x; 1.8644x over previous
import functools

import jax
import jax.numpy as jnp
from jax.experimental import pallas as pl
from jax.experimental.pallas import tpu as pltpu


def _fused_kernel(x_ref, c1_ref, c1b_ref, sl_ref, sr_ref, c2_ref, c2b_ref,
                  w1_ref, b1_ref, w2_ref, b2_ref, o_ref):
    B = x_ref.shape[1] // 2
    xfs = [x_ref[:, 0:B, :].reshape(28 * B, 28),
           x_ref[:, B:2 * B, :].reshape(28 * B, 28)]

    lhs1 = [jnp.concatenate([xf[ki * B:(ki + 24) * B] for ki in range(5)],
                            axis=-1) for xf in xfs]
    y1 = [jnp.maximum(jnp.dot(l, c1_ref[...],
                              preferred_element_type=jnp.float32)
                      + c1b_ref[...], 0.0).astype(jnp.bfloat16)
          for l in lhs1]

    mhw = []
    for y in y1:
        mh = jnp.maximum(y[0:23 * B], y[B:24 * B])
        mhw.append(jnp.maximum(mh[:, 0:239], mh[:, 1:240]))

    n4 = [jnp.dot(m, sr_ref[...], preferred_element_type=jnp.float32)
          .astype(jnp.bfloat16).reshape(23, B, 120)
          for m in mhw]

    p = [jnp.concatenate(
        [sum(n[h] * sl_ref[i, h].astype(jnp.bfloat16) for h in range(23))
         for i in range(12)], axis=0) for n in n4]

    y2 = [jnp.maximum(
        c2b_ref[...]
        + jnp.dot(q[0:10 * B], c2_ref[0], preferred_element_type=jnp.float32)
        + jnp.dot(q[B:11 * B], c2_ref[1], preferred_element_type=jnp.float32)
        + jnp.dot(q[2 * B:12 * B], c2_ref[2],
                  preferred_element_type=jnp.float32),
        0.0).astype(jnp.bfloat16) for q in p]

    h1 = [jnp.maximum(
        b1_ref[...] + sum(
            jnp.dot(y[h * B:(h + 1) * B], w1_ref[h],
                    preferred_element_type=jnp.float32)
            for h in range(10)),
        0.0).astype(jnp.bfloat16) for y in y2]

    for half, hh in enumerate(h1):
        z = jnp.dot(hh, w2_ref[...], preferred_element_type=jnp.float32) \
            + b2_ref[...]
        m = jnp.max(z, axis=-1, keepdims=True)
        lse = jnp.log(jnp.sum(jnp.exp(z - m), axis=-1, keepdims=True)) + m
        o_ref[half * B:(half + 1) * B] = z - lse


@functools.partial(jax.jit, static_argnames=())
def kernel(x, conv1_band, conv1_bias, pool_sl, pool_sr, conv2_band,
           conv2_bias, fc1_w, fc1_b, fc2_w, fc2_b):
    N = x.shape[0]

    B = 128
    while N % B:
        B //= 2
    grid = N // B

    xt = x.reshape(N, 28, 28).astype(jnp.bfloat16).transpose(1, 0, 2)
    c1 = conv1_band.astype(jnp.bfloat16).reshape(140, 240)
    c1b = conv1_bias.astype(jnp.bfloat16)
    sr = pool_sr.astype(jnp.bfloat16)
    c2 = conv2_band.astype(jnp.bfloat16)
    w1 = fc1_w.reshape(10, 200, 500).astype(jnp.bfloat16)
    w2 = fc2_w.astype(jnp.bfloat16)

    return pl.pallas_call(
        _fused_kernel,
        out_shape=jax.ShapeDtypeStruct((N, 10), jnp.float32),
        grid=(grid,),
        in_specs=[
            pl.BlockSpec((28, B, 28), lambda b: (0, b, 0)),
            pl.BlockSpec((140, 240), lambda b: (0, 0)),
            pl.BlockSpec((1, 240), lambda b: (0, 0)),
            pl.BlockSpec(memory_space=pltpu.SMEM),
            pl.BlockSpec((239, 120), lambda b: (0, 0)),
            pl.BlockSpec((3, 120, 200), lambda b: (0, 0, 0)),
            pl.BlockSpec((1, 200), lambda b: (0, 0)),
            pl.BlockSpec((10, 200, 500), lambda b: (0, 0, 0)),
            pl.BlockSpec((1, 500), lambda b: (0, 0)),
            pl.BlockSpec((500, 10), lambda b: (0, 0)),
            pl.BlockSpec((1, 10), lambda b: (0, 0)),
        ],
        out_specs=pl.BlockSpec((B, 10), lambda b: (b, 0)),
        compiler_params=pltpu.CompilerParams(
            dimension_semantics=("parallel",)),
        cost_estimate=pl.CostEstimate(
            flops=N * (24 * 140 * 240 + 23 * 239 * 120 + 10 * 360 * 200
                       + 2000 * 500 + 500 * 10) * 2,
            transcendentals=N * 11,
            bytes_accessed=N * (784 * 2 + 40) + 2 * (140 * 240 + 239 * 120
                                                     + 360 * 200 + 2000 * 500
                                                     + 500 * 10),
        ),
    )(xt, c1, c1b, pool_sl, sr, c2, conv2_bias, w1, fc1_b,
      w2, fc2_b)


# speedup vs baseline: 2.2073x; 1.1839x over previous
import functools

import jax
import jax.numpy as jnp
from jax.experimental import pallas as pl
from jax.experimental.pallas import tpu as pltpu


def _fused_kernel(x_ref, c1_ref, c1b_ref, sl_ref, sr_ref, c2_ref, c2b_ref,
                  w1_ref, b1_ref, w2_ref, b2_ref, o_ref):
    B = x_ref.shape[1] // 2
    xfs = [x_ref[:, 0:B, :].reshape(28 * B, 28),
           x_ref[:, B:2 * B, :].reshape(28 * B, 28)]

    lhs1 = [jnp.concatenate([xf[ki * B:(ki + 24) * B] for ki in range(5)],
                            axis=-1) for xf in xfs]
    y1 = [jnp.maximum(jnp.dot(l, c1_ref[...],
                              preferred_element_type=jnp.float32)
                      + c1b_ref[...], 0.0).astype(jnp.bfloat16)
          for l in lhs1]

    mhw = []
    for y in y1:
        mh = jnp.maximum(y[0:23 * B], y[B:24 * B])
        mhw.append(jnp.maximum(mh[:, 0:239], mh[:, 1:240]))

    n4 = [jnp.dot(m, sr_ref[...], preferred_element_type=jnp.float32)
          .astype(jnp.bfloat16).reshape(23, B, 120)
          for m in mhw]

    p = [jnp.concatenate(
        [sum(n[h] * sl_ref[i, h].astype(jnp.bfloat16) for h in range(23))
         for i in range(12)], axis=0) for n in n4]

    y2 = [jnp.maximum(
        c2b_ref[...]
        + jnp.dot(q[0:10 * B], c2_ref[0], preferred_element_type=jnp.float32)
        + jnp.dot(q[B:11 * B], c2_ref[1], preferred_element_type=jnp.float32)
        + jnp.dot(q[2 * B:12 * B], c2_ref[2],
                  preferred_element_type=jnp.float32),
        0.0).astype(jnp.bfloat16) for q in p]

    h1 = [jnp.maximum(
        b1_ref[...] + sum(
            jnp.dot(y[h * B:(h + 1) * B], w1_ref[h],
                    preferred_element_type=jnp.float32)
            for h in range(10)),
        0.0).astype(jnp.bfloat16) for y in y2]

    for half, hh in enumerate(h1):
        z = jnp.dot(hh, w2_ref[...], preferred_element_type=jnp.float32) \
            + b2_ref[...]
        m = jnp.max(z, axis=-1, keepdims=True)
        lse = jnp.log(jnp.sum(jnp.exp(z - m), axis=-1, keepdims=True)) + m
        o_ref[half * B:(half + 1) * B] = z - lse


@functools.partial(jax.jit, static_argnames=())
def kernel(x, conv1_band, conv1_bias, pool_sl, pool_sr, conv2_band,
           conv2_bias, fc1_w, fc1_b, fc2_w, fc2_b):
    N = x.shape[0]

    B = 256
    while N % B:
        B //= 2
    grid = N // B

    xt = x.reshape(N, 28, 28).astype(jnp.bfloat16).transpose(1, 0, 2)
    c1 = conv1_band.astype(jnp.bfloat16).reshape(140, 240)
    c1b = conv1_bias.astype(jnp.bfloat16)
    sr = pool_sr.astype(jnp.bfloat16)
    c2 = conv2_band.astype(jnp.bfloat16)
    w1 = fc1_w.reshape(10, 200, 500).astype(jnp.bfloat16)
    w2 = fc2_w.astype(jnp.bfloat16)

    return pl.pallas_call(
        _fused_kernel,
        out_shape=jax.ShapeDtypeStruct((N, 10), jnp.float32),
        grid=(grid,),
        in_specs=[
            pl.BlockSpec((28, B, 28), lambda b: (0, b, 0)),
            pl.BlockSpec((140, 240), lambda b: (0, 0)),
            pl.BlockSpec((1, 240), lambda b: (0, 0)),
            pl.BlockSpec(memory_space=pltpu.SMEM),
            pl.BlockSpec((239, 120), lambda b: (0, 0)),
            pl.BlockSpec((3, 120, 200), lambda b: (0, 0, 0)),
            pl.BlockSpec((1, 200), lambda b: (0, 0)),
            pl.BlockSpec((10, 200, 500), lambda b: (0, 0, 0)),
            pl.BlockSpec((1, 500), lambda b: (0, 0)),
            pl.BlockSpec((500, 10), lambda b: (0, 0)),
            pl.BlockSpec((1, 10), lambda b: (0, 0)),
        ],
        out_specs=pl.BlockSpec((B, 10), lambda b: (b, 0)),
        compiler_params=pltpu.CompilerParams(
            dimension_semantics=("parallel",)),
        cost_estimate=pl.CostEstimate(
            flops=N * (24 * 140 * 240 + 23 * 239 * 120 + 10 * 360 * 200
                       + 2000 * 500 + 500 * 10) * 2,
            transcendentals=N * 11,
            bytes_accessed=N * (784 * 2 + 40) + 2 * (140 * 240 + 239 * 120
                                                     + 360 * 200 + 2000 * 500
                                                     + 500 * 10),
        ),
    )(xt, c1, c1b, pool_sl, sr, c2, conv2_bias, w1, fc1_b,
      w2, fc2_b)


# speedup vs baseline: 2.2477x; 1.0183x over previous
import functools

import jax
import jax.numpy as jnp
from jax.experimental import pallas as pl
from jax.experimental.pallas import tpu as pltpu


def _fused_kernel(x_ref, c1_ref, c1b_ref, sl_ref, sr_ref, c2_ref, c2b_ref,
                  w1_ref, b1_ref, w2_ref, b2_ref, o_ref):
    B = x_ref.shape[1] // 2
    xfs = [x_ref[:, 0:B, :].reshape(28 * B, 28),
           x_ref[:, B:2 * B, :].reshape(28 * B, 28)]

    lhs1 = [jnp.concatenate([xf[ki * B:(ki + 24) * B] for ki in range(5)],
                            axis=-1) for xf in xfs]
    y1 = [jnp.maximum(jnp.dot(l, c1_ref[...],
                              preferred_element_type=jnp.float32)
                      + c1b_ref[...], 0.0).astype(jnp.bfloat16)
          for l in lhs1]

    mhw = []
    for y in y1:
        mh = jnp.maximum(y[0:23 * B], y[B:24 * B])
        mhw.append(jnp.maximum(mh[:, 0:239], mh[:, 1:240]))

    n4 = [jnp.dot(m, sr_ref[...], preferred_element_type=jnp.float32)
          .astype(jnp.bfloat16).reshape(23, B, 120)
          for m in mhw]

    p = [jnp.concatenate(
        [sum(n[h] * sl_ref[i, h].astype(jnp.bfloat16) for h in range(23))
         for i in range(12)], axis=0) for n in n4]

    y2 = [jnp.maximum(
        c2b_ref[...]
        + jnp.dot(q[0:10 * B], c2_ref[0], preferred_element_type=jnp.float32)
        + jnp.dot(q[B:11 * B], c2_ref[1], preferred_element_type=jnp.float32)
        + jnp.dot(q[2 * B:12 * B], c2_ref[2],
                  preferred_element_type=jnp.float32),
        0.0).astype(jnp.bfloat16) for q in p]

    h1 = [jnp.maximum(
        b1_ref[...] + sum(
            jnp.dot(y[h * B:(h + 1) * B], w1_ref[h],
                    preferred_element_type=jnp.float32)
            for h in range(10)),
        0.0).astype(jnp.bfloat16) for y in y2]

    for half, hh in enumerate(h1):
        z = jnp.dot(hh, w2_ref[...], preferred_element_type=jnp.float32) \
            + b2_ref[...]
        m = jnp.max(z, axis=-1, keepdims=True)
        lse = jnp.log(jnp.sum(jnp.exp(z - m), axis=-1, keepdims=True)) + m
        o_ref[half * B:(half + 1) * B] = z - lse


@functools.partial(jax.jit, static_argnames=())
def kernel(x, conv1_band, conv1_bias, pool_sl, pool_sr, conv2_band,
           conv2_bias, fc1_w, fc1_b, fc2_w, fc2_b):
    N = x.shape[0]

    B = 512
    while N % B:
        B //= 2
    grid = N // B

    xt = x.reshape(N, 28, 28).astype(jnp.bfloat16).transpose(1, 0, 2)
    c1 = conv1_band.astype(jnp.bfloat16).reshape(140, 240)
    c1b = conv1_bias.astype(jnp.bfloat16)
    sr = pool_sr.astype(jnp.bfloat16)
    c2 = conv2_band.astype(jnp.bfloat16)
    w1 = fc1_w.reshape(10, 200, 500).astype(jnp.bfloat16)
    w2 = fc2_w.astype(jnp.bfloat16)

    return pl.pallas_call(
        _fused_kernel,
        out_shape=jax.ShapeDtypeStruct((N, 10), jnp.float32),
        grid=(grid,),
        in_specs=[
            pl.BlockSpec((28, B, 28), lambda b: (0, b, 0)),
            pl.BlockSpec((140, 240), lambda b: (0, 0)),
            pl.BlockSpec((1, 240), lambda b: (0, 0)),
            pl.BlockSpec(memory_space=pltpu.SMEM),
            pl.BlockSpec((239, 120), lambda b: (0, 0)),
            pl.BlockSpec((3, 120, 200), lambda b: (0, 0, 0)),
            pl.BlockSpec((1, 200), lambda b: (0, 0)),
            pl.BlockSpec((10, 200, 500), lambda b: (0, 0, 0)),
            pl.BlockSpec((1, 500), lambda b: (0, 0)),
            pl.BlockSpec((500, 10), lambda b: (0, 0)),
            pl.BlockSpec((1, 10), lambda b: (0, 0)),
        ],
        out_specs=pl.BlockSpec((B, 10), lambda b: (b, 0)),
        compiler_params=pltpu.CompilerParams(
            dimension_semantics=("parallel",)),
        cost_estimate=pl.CostEstimate(
            flops=N * (24 * 140 * 240 + 23 * 239 * 120 + 10 * 360 * 200
                       + 2000 * 500 + 500 * 10) * 2,
            transcendentals=N * 11,
            bytes_accessed=N * (784 * 2 + 40) + 2 * (140 * 240 + 239 * 120
                                                     + 360 * 200 + 2000 * 500
                                                     + 500 * 10),
        ),
    )(xt, c1, c1b, pool_sl, sr, c2, conv2_bias, w1, fc1_b,
      w2, fc2_b)
